# Initial kernel scaffold; baseline (speedup 1.0000x reference)
import jax
import jax.numpy as jnp
from jax.experimental import pallas as pl


def kernel(x_stay, x_diag, params, ei_d2s_src, ei_d2s_dst, ei_s2d_src, ei_s2d_dst):
    raise NotImplementedError("write your pallas kernel here")


# speedup vs baseline: 42.2629x; 42.2629x over previous
import functools

import jax
import jax.numpy as jnp
from jax import lax
from jax.experimental import pallas as pl
from jax.experimental.pallas import tpu as pltpu
from jax.experimental.pallas import tpu_sc as plsc

N_STAY = 50000
N_DIAG = 10000
E = 300000
F_IN = 128
HID = 64
H = 2
D = 32
NC = 3
NL = 2

N_TILES = 16
IC = 128
EP = 311296
CHUNKS_PER_TILE = EP // (N_TILES * IC)

NDP_STAY = 50048
NDP_DIAG = 10240


def _row_split(ndp):
    rpt = ndp // N_TILES
    cw = 8
    for d in range(8, 513, 8):
        if rpt % d == 0:
            cw = d
    return rpt, cw


def _mm_body(act, x_ref, w_ref, b_ref, o_ref):
    y = jnp.dot(x_ref[...], w_ref[...], preferred_element_type=jnp.float32)
    y = y + b_ref[...]
    if act:
        y = jnp.maximum(y, 0.0)
    o_ref[...] = y


def _mm(x, w, b, act=False, bn=1000):
    n, k = x.shape
    f = w.shape[1]
    assert n % bn == 0
    return pl.pallas_call(
        functools.partial(_mm_body, act),
        out_shape=jax.ShapeDtypeStruct((n, f), jnp.float32),
        grid=(n // bn,),
        in_specs=[
            pl.BlockSpec((bn, k), lambda i: (i, 0)),
            pl.BlockSpec((k, f), lambda i: (0, 0)),
            pl.BlockSpec((1, f), lambda i: (0, 0)),
        ],
        out_specs=pl.BlockSpec((bn, f), lambda i: (i, 0)),
    )(x, w, b.reshape(1, f))


def _post_body(m_ref, d_ref, g_ref, b_ref, o_ref):
    m = m_ref[...]
    den = d_ref[...]
    bn = m.shape[0]
    dd = jnp.concatenate(
        [jnp.broadcast_to(den[:, 0:1], (bn, D)),
         jnp.broadcast_to(den[:, 1:2], (bn, D))], axis=-1)
    v = jnp.maximum(m / (dd + 1e-16), 0.0)
    mu = jnp.mean(v, axis=-1, keepdims=True)
    var = jnp.mean((v - mu) ** 2, axis=-1, keepdims=True)
    o_ref[...] = (v - mu) * lax.rsqrt(var + 1e-5) * g_ref[...] + b_ref[...]


def _post(msg, den, g, b, bn=1000):
    n = msg.shape[0]
    assert n % bn == 0
    return pl.pallas_call(
        _post_body,
        out_shape=jax.ShapeDtypeStruct((n, HID), jnp.float32),
        grid=(n // bn,),
        in_specs=[
            pl.BlockSpec((bn, HID), lambda i: (i, 0)),
            pl.BlockSpec((bn, H), lambda i: (i, 0)),
            pl.BlockSpec((1, HID), lambda i: (0, 0)),
            pl.BlockSpec((1, HID), lambda i: (0, 0)),
        ],
        out_specs=pl.BlockSpec((bn, HID), lambda i: (i, 0)),
    )(msg, den, g.reshape(1, HID), b.reshape(1, HID))


def _sc_conv_body(ns, ndp, rpt, cw,
                  zflat, alsrc, aldst, srce, dste, zrows0, zden0,
                  msg_out, den_out,
                  srcb, dstb, gidxb, gdstb, alsb, aldb, exc,
                  zrow, msgb, bounce, denb, sem, accum, dena):
    c = lax.axis_index("c")
    s = lax.axis_index("s")

    pltpu.sync_copy(zrows0, bounce)
    pltpu.sync_copy(zden0, denb)
    base = s * rpt
    for k in range(rpt // cw):
        pltpu.sync_copy(bounce, accum.at[pl.ds(base + k * cw, cw)])
    pltpu.sync_copy(denb, dena.at[pl.ds(base, rpt)])
    plsc.subcore_barrier()

    cns = c * ns
    cnd = c * ndp

    def chunk(i, carry):
        off = s * (CHUNKS_PER_TILE * IC) + i * IC
        pltpu.sync_copy(srce.at[pl.ds(off, IC)], srcb)
        pltpu.sync_copy(dste.at[pl.ds(off, IC)], dstb)
        for h in range(IC // 16):
            sv = srcb[pl.ds(h * 16, 16)]
            dv = dstb[pl.ds(h * 16, 16)]
            gidxb[pl.ds(h * 16, 16)] = sv + cns
            gdstb[pl.ds(h * 16, 16)] = dv + cnd
        pltpu.async_copy(alsrc.at[gidxb], alsb, sem).wait()
        pltpu.async_copy(aldst.at[gdstb], aldb, sem).wait()
        pltpu.async_copy(zflat.at[gidxb], zrow, sem).wait()
        exvals = []
        for h in range(IC // 16):
            av = alsb[pl.ds(h * 16, 16)] + aldb[pl.ds(h * 16, 16)]
            av = jnp.where(av >= 0, av, av * 0.2)
            ex = jnp.exp(av)
            exvals.append(ex)
            exc[pl.ds(h * 16, 16)] = ex
        for e in range(IC):
            exs = exvals[e // 16][e % 16]
            msgb[e, pl.ds(0, 16)] = zrow[e, pl.ds(0, 16)] * exs
            msgb[e, pl.ds(16, 16)] = zrow[e, pl.ds(16, 16)] * exs
        pltpu.sync_copy(msgb, accum.at[dstb], add=True)
        pltpu.sync_copy(exc, dena.at[dstb], add=True)
        return carry

    lax.fori_loop(0, CHUNKS_PER_TILE, chunk, 0)

    plsc.subcore_barrier()

    for k in range(rpt // cw):
        r = base + k * cw
        pltpu.sync_copy(accum.at[pl.ds(r, cw)], bounce)
        pltpu.sync_copy(bounce, msg_out.at[pl.ds(c * ndp + r, cw)])
    pltpu.sync_copy(dena.at[pl.ds(base, rpt)], denb)
    pltpu.sync_copy(denb, den_out.at[pl.ds(c * ndp + base, rpt)])


def _sc_conv(zflat, alsrc, aldst_p, src_p, dst_p, ns, ndp):
    rpt, cw = _row_split(ndp)
    mesh = plsc.VectorSubcoreMesh(core_axis_name="c", subcore_axis_name="s",
                                  num_cores=2, num_subcores=N_TILES)
    fn = pl.kernel(
        functools.partial(_sc_conv_body, ns, ndp, rpt, cw),
        out_type=(
            jax.ShapeDtypeStruct((2 * ndp, D), jnp.float32),
            jax.ShapeDtypeStruct((2 * ndp,), jnp.float32),
        ),
        mesh=mesh,
        compiler_params=pltpu.CompilerParams(needs_layout_passes=False,
                                             use_tc_tiling_on_sc=False),
        scratch_types=[
            pltpu.VMEM((IC,), jnp.int32),
            pltpu.VMEM((IC,), jnp.int32),
            pltpu.VMEM((IC,), jnp.int32),
            pltpu.VMEM((IC,), jnp.int32),
            pltpu.VMEM((IC,), jnp.float32),
            pltpu.VMEM((IC,), jnp.float32),
            pltpu.VMEM((IC,), jnp.float32),
            pltpu.VMEM((IC, D), jnp.float32),
            pltpu.VMEM((IC, D), jnp.float32),
            pltpu.VMEM((cw, D), jnp.float32),
            pltpu.VMEM((rpt,), jnp.float32),
            pltpu.SemaphoreType.DMA,
            pltpu.VMEM_SHARED((ndp, D), jnp.float32),
            pltpu.VMEM_SHARED((ndp,), jnp.float32),
        ],
    )
    zrows0 = jnp.zeros((cw, D), jnp.float32)
    zden0 = jnp.zeros((rpt,), jnp.float32)
    msg, den = fn(zflat, alsrc.reshape(-1), aldst_p.reshape(-1),
                  src_p, dst_p, zrows0, zden0)
    return msg.reshape(2, ndp, D), den.reshape(2, ndp)


def _block_attn_mat(a):
    z = jnp.zeros((D, 1), jnp.float32)
    return jnp.block([[a[0][:, None], z], [z, a[1][:, None]]])


def _split_cat(cat, n):
    z = cat[:, :HID].reshape(n, H, D).transpose(1, 0, 2).reshape(H * n, D)
    al_s = cat[:, HID:HID + 2].T
    al_d = cat[:, HID + 2:HID + 4].T
    return z, al_s, al_d


def _pad_al(al, ndp):
    return jnp.concatenate(
        [al, jnp.zeros((H, ndp - al.shape[1]), jnp.float32)], axis=1)


def kernel(x_stay, x_diag, params, ei_d2s_src, ei_d2s_dst, ei_s2d_src, ei_s2d_dst):
    p = params
    pad = EP - E
    e1s = jnp.concatenate([ei_d2s_src, jnp.zeros((pad,), jnp.int32)])
    e1d = jnp.concatenate([ei_d2s_dst, jnp.full((pad,), N_STAY, jnp.int32)])
    e2s = jnp.concatenate([ei_s2d_src, jnp.zeros((pad,), jnp.int32)])
    e2d = jnp.concatenate([ei_s2d_dst, jnp.full((pad,), N_DIAG, jnp.int32)])

    h_stay = _mm(x_stay, p["in_stay_W"], p["in_stay_b"], act=True)
    h_diag = _mm(x_diag, p["in_diag_W"], p["in_diag_b"], act=True)

    for l in range(NL):
        a_src_d2s = _block_attn_mat(p[f"l{l}_asrc_d2s"])
        a_dst_d2s = _block_attn_mat(p[f"l{l}_adst_d2s"])
        a_src_s2d = _block_attn_mat(p[f"l{l}_asrc_s2d"])
        a_dst_s2d = _block_attn_mat(p[f"l{l}_adst_s2d"])

        w_d, b_d = p[f"l{l}_proj_diag_W"], p[f"l{l}_proj_diag_b"]
        w_s, b_s = p[f"l{l}_proj_stay_W"], p[f"l{l}_proj_stay_b"]
        wcat_d = jnp.concatenate([w_d, w_d @ a_src_d2s, w_d @ a_dst_s2d], axis=1)
        bcat_d = jnp.concatenate([b_d, b_d @ a_src_d2s, b_d @ a_dst_s2d])
        wcat_s = jnp.concatenate([w_s, w_s @ a_src_s2d, w_s @ a_dst_d2s], axis=1)
        bcat_s = jnp.concatenate([b_s, b_s @ a_src_s2d, b_s @ a_dst_d2s])

        cat_d = _mm(h_diag, wcat_d, bcat_d)
        cat_s = _mm(h_stay, wcat_s, bcat_s)
        z_diag, alsrc_d2s, aldst_s2d = _split_cat(cat_d, N_DIAG)
        z_stay, alsrc_s2d, aldst_d2s = _split_cat(cat_s, N_STAY)

        msg_s, den_s = _sc_conv(z_diag, alsrc_d2s, _pad_al(aldst_d2s, NDP_STAY),
                                e1s, e1d, N_DIAG, NDP_STAY)
        h_stay = _post(
            msg_s[:, :N_STAY, :].transpose(1, 0, 2).reshape(N_STAY, HID),
            den_s[:, :N_STAY].T, p[f"l{l}_ln_g"], p[f"l{l}_ln_b"])
        if l + 1 < NL:
            msg_d, den_d = _sc_conv(z_stay, alsrc_s2d, _pad_al(aldst_s2d, NDP_DIAG),
                                    e2s, e2d, N_STAY, NDP_DIAG)
            h_diag = _post(
                msg_d[:, :N_DIAG, :].transpose(1, 0, 2).reshape(N_DIAG, HID),
                den_d[:, :N_DIAG].T, p[f"l{l}_ln_g"], p[f"l{l}_ln_b"])

    return _mm(h_stay, p["clf_W"], p["clf_b"])


# speedup vs baseline: 73.2353x; 1.7329x over previous
import functools

import jax
import jax.numpy as jnp
from jax import lax
from jax.experimental import pallas as pl
from jax.experimental.pallas import tpu as pltpu
from jax.experimental.pallas import tpu_sc as plsc

N_STAY = 50000
N_DIAG = 10000
E = 300000
F_IN = 128
HID = 64
H = 2
D = 32
NC = 3
NL = 2

N_TILES = 16
IC = 128
CHUNKS_PER_TILE = 156
EP = N_TILES * CHUNKS_PER_TILE * IC

NDP_STAY = 50048
NDP_DIAG = 10240


def _row_split(ndp):
    rpt = ndp // N_TILES
    cw = 8
    for d in range(8, 137, 8):
        if rpt % d == 0:
            cw = d
    return rpt, cw


def _mm_body(act, x_ref, w_ref, b_ref, o_ref):
    y = jnp.dot(x_ref[...], w_ref[...], preferred_element_type=jnp.float32)
    y = y + b_ref[...]
    if act:
        y = jnp.maximum(y, 0.0)
    o_ref[...] = y


def _mm(x, w, b, act=False, bn=1000):
    n, k = x.shape
    f = w.shape[1]
    assert n % bn == 0
    return pl.pallas_call(
        functools.partial(_mm_body, act),
        out_shape=jax.ShapeDtypeStruct((n, f), jnp.float32),
        grid=(n // bn,),
        in_specs=[
            pl.BlockSpec((bn, k), lambda i: (i, 0)),
            pl.BlockSpec((k, f), lambda i: (0, 0)),
            pl.BlockSpec((1, f), lambda i: (0, 0)),
        ],
        out_specs=pl.BlockSpec((bn, f), lambda i: (i, 0)),
    )(x, w, b.reshape(1, f))


def _post_body(m_ref, d_ref, g_ref, b_ref, o_ref):
    m = m_ref[...]
    den = d_ref[...]
    bn = m.shape[0]
    dd = jnp.concatenate(
        [jnp.broadcast_to(den[:, 0:1], (bn, D)),
         jnp.broadcast_to(den[:, 1:2], (bn, D))], axis=-1)
    v = jnp.maximum(m / (dd + 1e-16), 0.0)
    mu = jnp.mean(v, axis=-1, keepdims=True)
    var = jnp.mean((v - mu) ** 2, axis=-1, keepdims=True)
    o_ref[...] = (v - mu) * lax.rsqrt(var + 1e-5) * g_ref[...] + b_ref[...]


def _post(msg, den, g, b, bn=1000):
    n = msg.shape[0]
    assert n % bn == 0
    return pl.pallas_call(
        _post_body,
        out_shape=jax.ShapeDtypeStruct((n, HID), jnp.float32),
        grid=(n // bn,),
        in_specs=[
            pl.BlockSpec((bn, HID), lambda i: (i, 0)),
            pl.BlockSpec((bn, H), lambda i: (i, 0)),
            pl.BlockSpec((1, HID), lambda i: (0, 0)),
            pl.BlockSpec((1, HID), lambda i: (0, 0)),
        ],
        out_specs=pl.BlockSpec((bn, HID), lambda i: (i, 0)),
    )(msg, den, g.reshape(1, HID), b.reshape(1, HID))


def _sc_conv_body(ns, ndp, rpt, cw,
                  zflat, alsrc, aldst, srce, dste, zrows0, zden0,
                  msg_out, den_out,
                  eb_s, eb_d, gidxb, gdstb, alsb, aldb, exc,
                  zrow, msgb, bounce, denb, semi, semg, accum, dena):
    c = lax.axis_index("c")
    s = lax.axis_index("s")
    nchunk = CHUNKS_PER_TILE
    tbase = s * (nchunk * IC)
    cns = c * ns
    cnd = c * ndp

    def issue_idx(i, b):
        off = tbase + jnp.minimum(i, nchunk - 1) * IC
        pltpu.async_copy(srce.at[pl.ds(off, IC)], eb_s.at[b], semi.at[b])
        pltpu.async_copy(dste.at[pl.ds(off, IC)], eb_d.at[b], semi.at[b])

    def wait_idx(b):
        pltpu.make_async_copy(srce.at[pl.ds(0, IC)], eb_s.at[b], semi.at[b]).wait()
        pltpu.make_async_copy(dste.at[pl.ds(0, IC)], eb_d.at[b], semi.at[b]).wait()

    def build(b):
        for h in range(IC // 16):
            sv = eb_s[b, pl.ds(h * 16, 16)]
            dv = eb_d[b, pl.ds(h * 16, 16)]
            gidxb[b, pl.ds(h * 16, 16)] = sv + cns
            gdstb[b, pl.ds(h * 16, 16)] = dv + cnd

    def issue_gathers(b):
        pltpu.async_copy(alsrc.at[gidxb.at[b]], alsb.at[b], semg.at[b])
        pltpu.async_copy(aldst.at[gdstb.at[b]], aldb.at[b], semg.at[b])
        pltpu.async_copy(zflat.at[gidxb.at[b]], zrow.at[b], semg.at[b])

    def wait_gathers(b):
        pltpu.make_async_copy(alsrc.at[gidxb.at[b]], alsb.at[b], semg.at[b]).wait()
        pltpu.make_async_copy(aldst.at[gdstb.at[b]], aldb.at[b], semg.at[b]).wait()
        pltpu.make_async_copy(zflat.at[gidxb.at[b]], zrow.at[b], semg.at[b]).wait()

    def compute_scatter(b):
        exvals = []
        for h in range(IC // 16):
            av = alsb[b, pl.ds(h * 16, 16)] + aldb[b, pl.ds(h * 16, 16)]
            av = jnp.where(av >= 0, av, av * 0.2)
            ex = jnp.exp(av)
            exvals.append(ex)
            exc[pl.ds(h * 16, 16)] = ex
        for e in range(IC):
            exs = exvals[e // 16][e % 16]
            msgb[e, pl.ds(0, 16)] = zrow[b, e, pl.ds(0, 16)] * exs
            msgb[e, pl.ds(16, 16)] = zrow[b, e, pl.ds(16, 16)] * exs
        pltpu.sync_copy(msgb, accum.at[eb_d.at[b]], add=True)
        pltpu.sync_copy(exc, dena.at[eb_d.at[b]], add=True)

    issue_idx(0, 0)
    issue_idx(1, 1)

    pltpu.sync_copy(zrows0, bounce)
    pltpu.sync_copy(zden0, denb)
    base = s * rpt
    for k in range(rpt // cw):
        pltpu.sync_copy(bounce, accum.at[pl.ds(base + k * cw, cw)])
    pltpu.sync_copy(denb, dena.at[pl.ds(base, rpt)])
    plsc.subcore_barrier()

    wait_idx(0)
    build(0)
    issue_gathers(0)

    def body(k, carry):
        t = k * 3
        for b in range(3):
            i = t + b
            s_next = (b + 1) % 3
            s_idx = (b + 2) % 3
            wait_idx(s_next)
            build(s_next)
            issue_gathers(s_next)
            issue_idx(i + 2, s_idx)
            wait_gathers(b)
            compute_scatter(b)
        return carry

    lax.fori_loop(0, nchunk // 3, body, 0)

    wait_gathers(nchunk % 3)
    wait_idx((nchunk + 1) % 3)

    plsc.subcore_barrier()

    for k in range(rpt // cw):
        r = base + k * cw
        pltpu.sync_copy(accum.at[pl.ds(r, cw)], bounce)
        pltpu.sync_copy(bounce, msg_out.at[pl.ds(c * ndp + r, cw)])
    pltpu.sync_copy(dena.at[pl.ds(base, rpt)], denb)
    pltpu.sync_copy(denb, den_out.at[pl.ds(c * ndp + base, rpt)])


def _sc_conv(zflat, alsrc, aldst_p, src_p, dst_p, ns, ndp):
    rpt, cw = _row_split(ndp)
    mesh = plsc.VectorSubcoreMesh(core_axis_name="c", subcore_axis_name="s",
                                  num_cores=2, num_subcores=N_TILES)
    fn = pl.kernel(
        functools.partial(_sc_conv_body, ns, ndp, rpt, cw),
        out_type=(
            jax.ShapeDtypeStruct((2 * ndp, D), jnp.float32),
            jax.ShapeDtypeStruct((2 * ndp,), jnp.float32),
        ),
        mesh=mesh,
        compiler_params=pltpu.CompilerParams(needs_layout_passes=False,
                                             use_tc_tiling_on_sc=False),
        scratch_types=[
            pltpu.VMEM((3, IC), jnp.int32),
            pltpu.VMEM((3, IC), jnp.int32),
            pltpu.VMEM((3, IC), jnp.int32),
            pltpu.VMEM((3, IC), jnp.int32),
            pltpu.VMEM((3, IC), jnp.float32),
            pltpu.VMEM((3, IC), jnp.float32),
            pltpu.VMEM((IC,), jnp.float32),
            pltpu.VMEM((3, IC, D), jnp.float32),
            pltpu.VMEM((IC, D), jnp.float32),
            pltpu.VMEM((cw, D), jnp.float32),
            pltpu.VMEM((rpt,), jnp.float32),
            pltpu.SemaphoreType.DMA((3,)),
            pltpu.SemaphoreType.DMA((3,)),
            pltpu.VMEM_SHARED((ndp, D), jnp.float32),
            pltpu.VMEM_SHARED((ndp,), jnp.float32),
        ],
    )
    zrows0 = jnp.zeros((cw, D), jnp.float32)
    zden0 = jnp.zeros((rpt,), jnp.float32)
    msg, den = fn(zflat, alsrc.reshape(-1), aldst_p.reshape(-1),
                  src_p, dst_p, zrows0, zden0)
    return msg.reshape(2, ndp, D), den.reshape(2, ndp)


def _block_attn_mat(a):
    z = jnp.zeros((D, 1), jnp.float32)
    return jnp.block([[a[0][:, None], z], [z, a[1][:, None]]])


def _split_cat(cat, n):
    z = cat[:, :HID].reshape(n, H, D).transpose(1, 0, 2).reshape(H * n, D)
    al_s = cat[:, HID:HID + 2].T
    al_d = cat[:, HID + 2:HID + 4].T
    return z, al_s, al_d


def _pad_al(al, ndp):
    return jnp.concatenate(
        [al, jnp.zeros((H, ndp - al.shape[1]), jnp.float32)], axis=1)


def kernel(x_stay, x_diag, params, ei_d2s_src, ei_d2s_dst, ei_s2d_src, ei_s2d_dst):
    p = params
    pad = EP - E
    e1s = jnp.concatenate([ei_d2s_src, jnp.zeros((pad,), jnp.int32)])
    e1d = jnp.concatenate([ei_d2s_dst, jnp.full((pad,), N_STAY, jnp.int32)])
    e2s = jnp.concatenate([ei_s2d_src, jnp.zeros((pad,), jnp.int32)])
    e2d = jnp.concatenate([ei_s2d_dst, jnp.full((pad,), N_DIAG, jnp.int32)])

    h_stay = _mm(x_stay, p["in_stay_W"], p["in_stay_b"], act=True)
    h_diag = _mm(x_diag, p["in_diag_W"], p["in_diag_b"], act=True)

    for l in range(NL):
        a_src_d2s = _block_attn_mat(p[f"l{l}_asrc_d2s"])
        a_dst_d2s = _block_attn_mat(p[f"l{l}_adst_d2s"])
        a_src_s2d = _block_attn_mat(p[f"l{l}_asrc_s2d"])
        a_dst_s2d = _block_attn_mat(p[f"l{l}_adst_s2d"])

        w_d, b_d = p[f"l{l}_proj_diag_W"], p[f"l{l}_proj_diag_b"]
        w_s, b_s = p[f"l{l}_proj_stay_W"], p[f"l{l}_proj_stay_b"]
        wcat_d = jnp.concatenate([w_d, w_d @ a_src_d2s, w_d @ a_dst_s2d], axis=1)
        bcat_d = jnp.concatenate([b_d, b_d @ a_src_d2s, b_d @ a_dst_s2d])
        wcat_s = jnp.concatenate([w_s, w_s @ a_src_s2d, w_s @ a_dst_d2s], axis=1)
        bcat_s = jnp.concatenate([b_s, b_s @ a_src_s2d, b_s @ a_dst_d2s])

        cat_d = _mm(h_diag, wcat_d, bcat_d)
        cat_s = _mm(h_stay, wcat_s, bcat_s)
        z_diag, alsrc_d2s, aldst_s2d = _split_cat(cat_d, N_DIAG)
        z_stay, alsrc_s2d, aldst_d2s = _split_cat(cat_s, N_STAY)

        msg_s, den_s = _sc_conv(z_diag, alsrc_d2s, _pad_al(aldst_d2s, NDP_STAY),
                                e1s, e1d, N_DIAG, NDP_STAY)
        h_stay = _post(
            msg_s[:, :N_STAY, :].transpose(1, 0, 2).reshape(N_STAY, HID),
            den_s[:, :N_STAY].T, p[f"l{l}_ln_g"], p[f"l{l}_ln_b"])
        if l + 1 < NL:
            msg_d, den_d = _sc_conv(z_stay, alsrc_s2d, _pad_al(aldst_s2d, NDP_DIAG),
                                    e2s, e2d, N_STAY, NDP_DIAG)
            h_diag = _post(
                msg_d[:, :N_DIAG, :].transpose(1, 0, 2).reshape(N_DIAG, HID),
                den_d[:, :N_DIAG].T, p[f"l{l}_ln_g"], p[f"l{l}_ln_b"])

    return _mm(h_stay, p["clf_W"], p["clf_b"])
